# Initial kernel scaffold; baseline (speedup 1.0000x reference)
import jax
import jax.numpy as jnp
from jax.experimental import pallas as pl


def kernel(x, Wq, Wk, Wv, Wo, ln1_g, ln1_b, W1, b1, W2, b2, ln2_g, ln2_b):
    raise NotImplementedError("write your pallas kernel here")


# speedup vs baseline: 1.9085x; 1.9085x over previous
import jax
import jax.numpy as jnp
from jax.experimental import pallas as pl

N = 50000
D = 256
D_INNER = 1024
BN1 = 2000
BN2 = 1000


def _phi(z):
    return jnp.where(z > 0, z + 1.0, jnp.exp(z))


def _ln(xb, g, b, eps=1e-5):
    mu = jnp.mean(xb, axis=-1, keepdims=True)
    var = jnp.mean((xb - mu) ** 2, axis=-1, keepdims=True)
    return (xb - mu) * jax.lax.rsqrt(var + eps) * g + b


def _pass1(x_ref, wk_ref, wv_ref, g_ref, b_ref, kv_ref, ksum_ref):
    i = pl.program_id(0)
    h = _ln(x_ref[...], g_ref[...], b_ref[...])
    k = jnp.dot(h, wk_ref[...], preferred_element_type=jnp.float32)
    v = jnp.dot(h, wv_ref[...], preferred_element_type=jnp.float32)
    phik = _phi(k)
    pkv = jax.lax.dot_general(
        phik, v, (((0,), (0,)), ((), ())),
        preferred_element_type=jnp.float32)
    pksum = jnp.sum(phik, axis=0, keepdims=True)

    @pl.when(i == 0)
    def _():
        kv_ref[...] = jnp.zeros_like(kv_ref)
        ksum_ref[...] = jnp.zeros_like(ksum_ref)

    kv_ref[...] += pkv
    ksum_ref[...] += pksum


def _pass2(x_ref, wq_ref, wo_ref, kv_ref, ksum_ref, g1_ref, b1_ref,
           w1_ref, bb1_ref, w2_ref, bb2_ref, g2_ref, b2_ref, out_ref):
    xb = x_ref[...]
    h = _ln(xb, g1_ref[...], b1_ref[...])
    q = jnp.dot(h, wq_ref[...], preferred_element_type=jnp.float32)
    phiq = _phi(q)
    num = jnp.dot(phiq, kv_ref[...], preferred_element_type=jnp.float32)
    den = jnp.sum(phiq * ksum_ref[...], axis=1, keepdims=True) + 1e-6
    attn = jnp.dot(num / den, wo_ref[...], preferred_element_type=jnp.float32)
    x2 = xb + attn
    h2 = _ln(x2, g2_ref[...], b2_ref[...])
    inner = jax.nn.gelu(
        jnp.dot(h2, w1_ref[...], preferred_element_type=jnp.float32)
        + bb1_ref[...])
    mlp = jnp.dot(inner, w2_ref[...], preferred_element_type=jnp.float32)
    out_ref[...] = x2 + mlp + bb2_ref[...]


def kernel(x, Wq, Wk, Wv, Wo, ln1_g, ln1_b, W1, b1, W2, b2, ln2_g, ln2_b):
    g1 = ln1_g.reshape(1, D)
    bt1 = ln1_b.reshape(1, D)
    g2 = ln2_g.reshape(1, D)
    bt2 = ln2_b.reshape(1, D)
    bb1 = b1.reshape(1, D_INNER)
    bb2 = b2.reshape(1, D)

    full = lambda shape: pl.BlockSpec(shape, lambda i: (0,) * len(shape))

    kv, ksum = pl.pallas_call(
        _pass1,
        grid=(N // BN1,),
        in_specs=[
            pl.BlockSpec((BN1, D), lambda i: (i, 0)),
            full((D, D)), full((D, D)), full((1, D)), full((1, D)),
        ],
        out_specs=[full((D, D)), full((1, D))],
        out_shape=[
            jax.ShapeDtypeStruct((D, D), jnp.float32),
            jax.ShapeDtypeStruct((1, D), jnp.float32),
        ],
    )(x, Wk, Wv, g1, bt1)

    out = pl.pallas_call(
        _pass2,
        grid=(N // BN2,),
        in_specs=[
            pl.BlockSpec((BN2, D), lambda i: (i, 0)),
            full((D, D)), full((D, D)), full((D, D)), full((1, D)),
            full((1, D)), full((1, D)),
            full((D, D_INNER)), full((1, D_INNER)),
            full((D_INNER, D)), full((1, D)),
            full((1, D)), full((1, D)),
        ],
        out_specs=pl.BlockSpec((BN2, D), lambda i: (i, 0)),
        out_shape=jax.ShapeDtypeStruct((N, D), jnp.float32),
    )(x, Wq, Wo, kv, ksum, g1, bt1, W1, bb1, W2, bb2, g2, bt2)
    return out
